# baseline (speedup 1.0000x reference)
import functools

import jax
import jax.numpy as jnp
from jax import lax
from jax.experimental import pallas as pl
from jax.experimental.pallas import tpu as pltpu, tpu_sc as plsc

_LANES = 16
_W = 16


def _round_up(x, m):
    return (x + m - 1) // m * m


@functools.partial(jax.jit, static_argnames=())
def _sc_gather(b_idx, l_idx, x_idx, boxes_t, scores2d):
    info = plsc.get_sparse_core_info()
    nw = info.num_cores * info.num_subcores
    padded = b_idx.shape[0]
    chunk = padded // nw
    n_grp = chunk // _LANES
    n_batch = boxes_t.shape[0]

    mesh = plsc.VectorSubcoreMesh(core_axis_name="c", subcore_axis_name="s")

    @functools.partial(
        pl.kernel,
        mesh=mesh,
        out_type=jax.ShapeDtypeStruct((8, padded), jnp.float32),
        compiler_params=pltpu.CompilerParams(needs_layout_passes=False),
        scratch_types=[
            pltpu.VMEM((chunk,), jnp.int32),
            pltpu.VMEM((chunk,), jnp.int32),
            pltpu.VMEM((chunk,), jnp.int32),
            pltpu.VMEM((_LANES, 8, 128), jnp.float32),
            pltpu.VMEM((_LANES, 8, 128), jnp.float32),
            pltpu.VMEM((_LANES, 4, 128), jnp.float32),
            pltpu.VMEM((_LANES, 4, 128), jnp.float32),
            pltpu.VMEM((8, chunk), jnp.float32),
            pltpu.SemaphoreType.DMA,
        ],
    )
    def body(b_hbm, l_hbm, x_hbm, boxes_hbm, scores_hbm, out_hbm,
             b_v, l_v, x_v, sw0_v, sw1_v, bw0_v, bw1_v, out_v, sem):
        sw_refs = (sw0_v, sw1_v)
        bw_refs = (bw0_v, bw1_v)
        wid = lax.axis_index("s") * info.num_cores + lax.axis_index("c")
        base = wid * chunk

        pltpu.sync_copy(b_hbm.at[pl.ds(base, chunk)], b_v)
        pltpu.sync_copy(l_hbm.at[pl.ds(base, chunk)], l_v)
        pltpu.sync_copy(x_hbm.at[pl.ds(base, chunk)], x_v)

        lane16 = lax.iota(jnp.int32, _LANES)

        def fire(g, slot):
            off = g * _LANES
            b16 = b_v[pl.ds(off, _LANES)]
            l16 = l_v[pl.ds(off, _LANES)]
            x16 = x_v[pl.ds(off, _LANES)]
            row16 = l16 * n_batch + b16
            rowa16 = lax.bitwise_and(row16, ~7)
            xa16 = lax.bitwise_and(x16, ~127)
            copies = []
            for k in range(_LANES):
                b = b16[k]
                rowa = pl.multiple_of(rowa16[k], 8)
                xa = pl.multiple_of(xa16[k], 128)
                copies.append(pltpu.async_copy(
                    scores_hbm.at[pl.ds(rowa, 8), pl.ds(xa, 128)],
                    sw_refs[slot].at[k], sem))
                copies.append(pltpu.async_copy(
                    boxes_hbm.at[b, :, pl.ds(xa, 128)],
                    bw_refs[slot].at[k], sem))
            out_v[0, pl.ds(off, _LANES)] = b16.astype(jnp.float32)
            out_v[6, pl.ds(off, _LANES)] = l16.astype(jnp.float32)
            return copies, b16, l16, x16, row16

        def extract(g, slot, copies, l16, x16, row16):
            off = g * _LANES
            for c in copies:
                c.wait()
            col = lax.bitwise_and(x16, 127)
            subrow = lax.bitwise_and(row16, 7)
            out_v[5, pl.ds(off, _LANES)] = plsc.load_gather(
                sw_refs[slot], [lane16, subrow, col])
            for c in range(4):
                cc = jnp.full((_LANES,), c, jnp.int32)
                out_v[1 + c, pl.ds(off, _LANES)] = plsc.load_gather(
                    bw_refs[slot], [lane16, cc, col])

        def pair(p, carry):
            g0 = p * 2
            ca, _, la, xa_, ra = fire(g0, 0)
            cb, _, lb2, xb, rb = fire(g0 + 1, 1)
            extract(g0, 0, ca, la, xa_, ra)
            extract(g0 + 1, 1, cb, lb2, xb, rb)
            return carry

        lax.fori_loop(0, n_grp // 2, pair, 0)

        pltpu.sync_copy(out_v, out_hbm.at[:, pl.ds(base, chunk)])

    return body(b_idx, l_idx, x_idx, boxes_t, scores2d)


def kernel(pred_boxes, pred_scores, selected_indexes):
    n_batch, n_anchors, n_box = pred_boxes.shape
    n_labels = pred_scores.shape[-1]
    n_rows = selected_indexes.shape[0]

    nw = 32
    padded = _round_up(n_rows, nw * _LANES)
    si = jnp.pad(selected_indexes, ((0, padded - n_rows), (0, 0)))
    b_idx = si[:, 0]
    l_idx = si[:, 1]
    x_idx = si[:, 2]

    scores2d = jnp.transpose(pred_scores, (2, 0, 1)).reshape(
        n_labels * n_batch, n_anchors)
    boxes_t = jnp.transpose(pred_boxes, (0, 2, 1))

    out_t = _sc_gather(b_idx, l_idx, x_idx, boxes_t, scores2d)
    return out_t[:7, :n_rows].T


# speedup vs baseline: 1.0515x; 1.0515x over previous
import functools

import jax
import jax.numpy as jnp
from jax import lax
from jax.experimental import pallas as pl
from jax.experimental.pallas import tpu as pltpu, tpu_sc as plsc

_LANES = 16
_W = 16


def _round_up(x, m):
    return (x + m - 1) // m * m


@functools.partial(jax.jit, static_argnames=())
def _sc_gather(b_idx, l_idx, x_idx, boxes_t, scores2d):
    info = plsc.get_sparse_core_info()
    nw = info.num_cores * info.num_subcores
    padded = b_idx.shape[0]
    chunk = padded // nw
    n_grp = chunk // _LANES
    n_batch = boxes_t.shape[0]

    mesh = plsc.VectorSubcoreMesh(core_axis_name="c", subcore_axis_name="s")

    @functools.partial(
        pl.kernel,
        mesh=mesh,
        out_type=jax.ShapeDtypeStruct((8, padded), jnp.float32),
        compiler_params=pltpu.CompilerParams(needs_layout_passes=False),
        scratch_types=[
            pltpu.VMEM((chunk,), jnp.int32),
            pltpu.VMEM((chunk,), jnp.int32),
            pltpu.VMEM((chunk,), jnp.int32),
            pltpu.VMEM((_LANES, 8, 128), jnp.float32),
            pltpu.VMEM((_LANES, 8, 128), jnp.float32),
            pltpu.VMEM((_LANES, 4, 128), jnp.float32),
            pltpu.VMEM((_LANES, 4, 128), jnp.float32),
            pltpu.VMEM((8, chunk), jnp.float32),
            pltpu.SemaphoreType.DMA,
        ],
    )
    def body(b_hbm, l_hbm, x_hbm, boxes_hbm, scores_hbm, out_hbm,
             b_v, l_v, x_v, sw0_v, sw1_v, bw0_v, bw1_v, out_v, sem):
        sw_refs = (sw0_v, sw1_v)
        bw_refs = (bw0_v, bw1_v)
        wid = lax.axis_index("s") * info.num_cores + lax.axis_index("c")
        base = wid * chunk

        pltpu.sync_copy(b_hbm.at[pl.ds(base, chunk)], b_v)
        pltpu.sync_copy(l_hbm.at[pl.ds(base, chunk)], l_v)
        pltpu.sync_copy(x_hbm.at[pl.ds(base, chunk)], x_v)

        lane16 = lax.iota(jnp.int32, _LANES)

        def fire(g, slot):
            off = g * _LANES
            b16 = b_v[pl.ds(off, _LANES)]
            l16 = l_v[pl.ds(off, _LANES)]
            x16 = x_v[pl.ds(off, _LANES)]
            row16 = l16 * n_batch + b16
            rowa16 = lax.bitwise_and(row16, ~7)
            xa16 = lax.bitwise_and(x16, ~127)
            copies = []
            for k in range(_LANES):
                b = b16[k]
                rowa = pl.multiple_of(rowa16[k], 8)
                xa = pl.multiple_of(xa16[k], 128)
                copies.append(pltpu.async_copy(
                    scores_hbm.at[pl.ds(rowa, 8), pl.ds(xa, 128)],
                    sw_refs[slot].at[k], sem))
                copies.append(pltpu.async_copy(
                    boxes_hbm.at[b, :, pl.ds(xa, 128)],
                    bw_refs[slot].at[k], sem))
            out_v[0, pl.ds(off, _LANES)] = b16.astype(jnp.float32)
            out_v[6, pl.ds(off, _LANES)] = l16.astype(jnp.float32)
            return copies, b16, l16, x16, row16

        def extract(g, slot, copies, l16, x16, row16):
            off = g * _LANES
            for c in copies:
                c.wait()
            col = lax.bitwise_and(x16, 127)
            subrow = lax.bitwise_and(row16, 7)
            out_v[5, pl.ds(off, _LANES)] = plsc.load_gather(
                sw_refs[slot], [lane16, subrow, col])
            for c in range(4):
                cc = jnp.full((_LANES,), c, jnp.int32)
                out_v[1 + c, pl.ds(off, _LANES)] = plsc.load_gather(
                    bw_refs[slot], [lane16, cc, col])

        def group(g, carry):
            ca, _, la, xa_, ra = fire(g, 0)
            extract(g, 0, ca, la, xa_, ra)
            return carry

        lax.fori_loop(0, n_grp, group, 0)

        pltpu.sync_copy(out_v, out_hbm.at[:, pl.ds(base, chunk)])

    return body(b_idx, l_idx, x_idx, boxes_t, scores2d)


def kernel(pred_boxes, pred_scores, selected_indexes):
    n_batch, n_anchors, n_box = pred_boxes.shape
    n_labels = pred_scores.shape[-1]
    n_rows = selected_indexes.shape[0]

    nw = 32
    padded = _round_up(n_rows, nw * _LANES)
    si = jnp.pad(selected_indexes, ((0, padded - n_rows), (0, 0)))
    b_idx = si[:, 0]
    l_idx = si[:, 1]
    x_idx = si[:, 2]

    scores2d = jnp.transpose(pred_scores, (2, 0, 1)).reshape(
        n_labels * n_batch, n_anchors)
    boxes_t = jnp.transpose(pred_boxes, (0, 2, 1))

    out_t = _sc_gather(b_idx, l_idx, x_idx, boxes_t, scores2d)
    return out_t[:7, :n_rows].T
